# baseline (speedup 1.0000x reference)
import jax
import jax.numpy as jnp
from jax.experimental import pallas as pl
import jax.experimental.pallas.tpu as pltpu

B = 64
C = 256
D = C * 4
NUM_EXPERTS = 4
NUM_CLASSES = 6

C_CHUNK = 128


def _pool_body(f0, f1, f2, f3, out):
    out[0, 0, :] = jnp.sum(f0[0], axis=1) * (1.0 / (56 * 56))
    out[0, 1, :] = jnp.sum(f1[0], axis=1) * (1.0 / (28 * 28))
    out[0, 2, :] = jnp.sum(f2[0], axis=1) * (1.0 / (14 * 14))
    out[0, 3, :] = jnp.sum(f3[0], axis=1) * (1.0 / (7 * 7))


def _head_body(pooled, wg, bg, we, be, out, gw_out):
    feat = pooled[...]
    gate = jax.lax.dot_general(
        feat, wg[...], (((1,), (0,)), ((), ())),
        preferred_element_type=jnp.float32) + bg[...]
    m = jnp.max(gate, axis=1, keepdims=True)
    ex = jnp.exp(gate - m)
    gw = ex / jnp.sum(ex, axis=1, keepdims=True)
    acc = jnp.zeros((B, NUM_CLASSES), dtype=jnp.float32)
    for e in range(NUM_EXPERTS):
        eo = jax.lax.dot_general(
            feat, we[e], (((1,), (0,)), ((), ())),
            preferred_element_type=jnp.float32) + be[e:e + 1, :]
        acc = acc + gw[:, e:e + 1] * eo
    out[...] = acc
    gw_out[...] = gw


def kernel(feature_0, feature_1, feature_2, feature_3, c_feature, t_feature,
           Wg, bg, We, be):
    del c_feature, t_feature
    f0 = feature_0.reshape(B, C, 56 * 56)
    f1 = feature_1.reshape(B, C, 28 * 28)
    f2 = feature_2.reshape(B, C, 14 * 14)
    f3 = feature_3.reshape(B, C, 7 * 7)
    pooled = pl.pallas_call(
        _pool_body,
        grid=(B,),
        in_specs=[
            pl.BlockSpec((1, C, 56 * 56), lambda b: (b, 0, 0)),
            pl.BlockSpec((1, C, 28 * 28), lambda b: (b, 0, 0)),
            pl.BlockSpec((1, C, 14 * 14), lambda b: (b, 0, 0)),
            pl.BlockSpec((1, C, 7 * 7), lambda b: (b, 0, 0)),
        ],
        out_specs=pl.BlockSpec((1, NUM_EXPERTS, C), lambda b: (b, 0, 0)),
        out_shape=jax.ShapeDtypeStruct((B, NUM_EXPERTS, C), jnp.float32),
        compiler_params=pltpu.CompilerParams(
            dimension_semantics=("parallel",)),
    )(f0, f1, f2, f3)

    feat = pooled.reshape(B, D)
    out, gw = pl.pallas_call(
        _head_body,
        in_specs=[
            pl.BlockSpec((B, D), lambda: (0, 0)),
            pl.BlockSpec(Wg.shape, lambda: (0, 0)),
            pl.BlockSpec((1, NUM_EXPERTS), lambda: (0, 0)),
            pl.BlockSpec(We.shape, lambda: (0, 0, 0)),
            pl.BlockSpec(be.shape, lambda: (0, 0)),
        ],
        out_specs=[
            pl.BlockSpec((B, NUM_CLASSES), lambda: (0, 0)),
            pl.BlockSpec((B, NUM_EXPERTS), lambda: (0, 0)),
        ],
        out_shape=[
            jax.ShapeDtypeStruct((B, NUM_CLASSES), jnp.float32),
            jax.ShapeDtypeStruct((B, NUM_EXPERTS), jnp.float32),
        ],
    )(feat, Wg, bg.reshape(1, NUM_EXPERTS), We, be)
    return (out, gw)


# speedup vs baseline: 1.0087x; 1.0087x over previous
import jax
import jax.numpy as jnp
from jax.experimental import pallas as pl
import jax.experimental.pallas.tpu as pltpu

B = 64
C = 256
D = C * 4
NUM_EXPERTS = 4
NUM_CLASSES = 6

C_CHUNK = 128


def _pool_body(f0a, f0b, f0c, f0d, f1a, f1b, f2, f3, out):
    out[0, 0, 0:64] = jnp.sum(f0a[0], axis=1) * (1.0 / (56 * 56))
    out[0, 0, 64:128] = jnp.sum(f0b[0], axis=1) * (1.0 / (56 * 56))
    out[0, 0, 128:192] = jnp.sum(f0c[0], axis=1) * (1.0 / (56 * 56))
    out[0, 0, 192:256] = jnp.sum(f0d[0], axis=1) * (1.0 / (56 * 56))
    out[0, 1, 0:128] = jnp.sum(f1a[0], axis=1) * (1.0 / (28 * 28))
    out[0, 1, 128:256] = jnp.sum(f1b[0], axis=1) * (1.0 / (28 * 28))
    out[0, 2, :] = jnp.sum(f2[0], axis=1) * (1.0 / (14 * 14))
    out[0, 3, :] = jnp.sum(f3[0], axis=1) * (1.0 / (7 * 7))


def _head_body(pooled, wg, bg, we, be, out, gw_out):
    feat = pooled[...]
    gate = jax.lax.dot_general(
        feat, wg[...], (((1,), (0,)), ((), ())),
        preferred_element_type=jnp.float32) + bg[...]
    m = jnp.max(gate, axis=1, keepdims=True)
    ex = jnp.exp(gate - m)
    gw = ex / jnp.sum(ex, axis=1, keepdims=True)
    acc = jnp.zeros((B, NUM_CLASSES), dtype=jnp.float32)
    for e in range(NUM_EXPERTS):
        eo = jax.lax.dot_general(
            feat, we[e], (((1,), (0,)), ((), ())),
            preferred_element_type=jnp.float32) + be[e:e + 1, :]
        acc = acc + gw[:, e:e + 1] * eo
    out[...] = acc
    gw_out[...] = gw


def kernel(feature_0, feature_1, feature_2, feature_3, c_feature, t_feature,
           Wg, bg, We, be):
    del c_feature, t_feature
    f0 = feature_0.reshape(B, C, 56 * 56)
    f1 = feature_1.reshape(B, C, 28 * 28)
    f2 = feature_2.reshape(B, C, 14 * 14)
    f3 = feature_3.reshape(B, C, 7 * 7)
    pooled = pl.pallas_call(
        _pool_body,
        grid=(B,),
        in_specs=[
            pl.BlockSpec((1, 64, 56 * 56), lambda b: (b, 0, 0)),
            pl.BlockSpec((1, 64, 56 * 56), lambda b: (b, 1, 0)),
            pl.BlockSpec((1, 64, 56 * 56), lambda b: (b, 2, 0)),
            pl.BlockSpec((1, 64, 56 * 56), lambda b: (b, 3, 0)),
            pl.BlockSpec((1, 128, 28 * 28), lambda b: (b, 0, 0)),
            pl.BlockSpec((1, 128, 28 * 28), lambda b: (b, 1, 0)),
            pl.BlockSpec((1, C, 14 * 14), lambda b: (b, 0, 0)),
            pl.BlockSpec((1, C, 7 * 7), lambda b: (b, 0, 0)),
        ],
        out_specs=pl.BlockSpec((1, NUM_EXPERTS, C), lambda b: (b, 0, 0)),
        out_shape=jax.ShapeDtypeStruct((B, NUM_EXPERTS, C), jnp.float32),
        compiler_params=pltpu.CompilerParams(
            dimension_semantics=("parallel",)),
    )(f0, f0, f0, f0, f1, f1, f2, f3)

    feat = pooled.reshape(B, D)
    out, gw = pl.pallas_call(
        _head_body,
        in_specs=[
            pl.BlockSpec((B, D), lambda: (0, 0)),
            pl.BlockSpec(Wg.shape, lambda: (0, 0)),
            pl.BlockSpec((1, NUM_EXPERTS), lambda: (0, 0)),
            pl.BlockSpec(We.shape, lambda: (0, 0, 0)),
            pl.BlockSpec(be.shape, lambda: (0, 0)),
        ],
        out_specs=[
            pl.BlockSpec((B, NUM_CLASSES), lambda: (0, 0)),
            pl.BlockSpec((B, NUM_EXPERTS), lambda: (0, 0)),
        ],
        out_shape=[
            jax.ShapeDtypeStruct((B, NUM_CLASSES), jnp.float32),
            jax.ShapeDtypeStruct((B, NUM_EXPERTS), jnp.float32),
        ],
    )(feat, Wg, bg.reshape(1, NUM_EXPERTS), We, be)
    return (out, gw)


# speedup vs baseline: 1.0334x; 1.0245x over previous
import jax
import jax.numpy as jnp
from jax.experimental import pallas as pl
import jax.experimental.pallas.tpu as pltpu

B = 64
C = 256
D = C * 4
NUM_EXPERTS = 4
NUM_CLASSES = 6

C_CHUNK = 128


BB = 4


def _pool_body(f0a, f0b, f0c, f0d, f1a, f1b, f2, f3, out):
    for i in range(BB):
        out[i, 0, 0:64] = jnp.sum(f0a[i], axis=1) * (1.0 / (56 * 56))
        out[i, 0, 64:128] = jnp.sum(f0b[i], axis=1) * (1.0 / (56 * 56))
        out[i, 0, 128:192] = jnp.sum(f0c[i], axis=1) * (1.0 / (56 * 56))
        out[i, 0, 192:256] = jnp.sum(f0d[i], axis=1) * (1.0 / (56 * 56))
        out[i, 1, 0:128] = jnp.sum(f1a[i], axis=1) * (1.0 / (28 * 28))
        out[i, 1, 128:256] = jnp.sum(f1b[i], axis=1) * (1.0 / (28 * 28))
        out[i, 2, :] = jnp.sum(f2[i], axis=1) * (1.0 / (14 * 14))
        out[i, 3, :] = jnp.sum(f3[i], axis=1) * (1.0 / (7 * 7))


def _head_body(pooled, wg, bg, we, be, out, gw_out):
    feat = pooled[...]
    gate = jax.lax.dot_general(
        feat, wg[...], (((1,), (0,)), ((), ())),
        preferred_element_type=jnp.float32) + bg[...]
    m = jnp.max(gate, axis=1, keepdims=True)
    ex = jnp.exp(gate - m)
    gw = ex / jnp.sum(ex, axis=1, keepdims=True)
    acc = jnp.zeros((B, NUM_CLASSES), dtype=jnp.float32)
    for e in range(NUM_EXPERTS):
        eo = jax.lax.dot_general(
            feat, we[e], (((1,), (0,)), ((), ())),
            preferred_element_type=jnp.float32) + be[e:e + 1, :]
        acc = acc + gw[:, e:e + 1] * eo
    out[...] = acc
    gw_out[...] = gw


def kernel(feature_0, feature_1, feature_2, feature_3, c_feature, t_feature,
           Wg, bg, We, be):
    del c_feature, t_feature
    f0 = feature_0.reshape(B, C, 56 * 56)
    f1 = feature_1.reshape(B, C, 28 * 28)
    f2 = feature_2.reshape(B, C, 14 * 14)
    f3 = feature_3.reshape(B, C, 7 * 7)
    pooled = pl.pallas_call(
        _pool_body,
        grid=(B // BB,),
        in_specs=[
            pl.BlockSpec((BB, 64, 56 * 56), lambda b: (b, 0, 0)),
            pl.BlockSpec((BB, 64, 56 * 56), lambda b: (b, 1, 0)),
            pl.BlockSpec((BB, 64, 56 * 56), lambda b: (b, 2, 0)),
            pl.BlockSpec((BB, 64, 56 * 56), lambda b: (b, 3, 0)),
            pl.BlockSpec((BB, 128, 28 * 28), lambda b: (b, 0, 0)),
            pl.BlockSpec((BB, 128, 28 * 28), lambda b: (b, 1, 0)),
            pl.BlockSpec((BB, C, 14 * 14), lambda b: (b, 0, 0)),
            pl.BlockSpec((BB, C, 7 * 7), lambda b: (b, 0, 0)),
        ],
        out_specs=pl.BlockSpec((BB, NUM_EXPERTS, C), lambda b: (b, 0, 0)),
        out_shape=jax.ShapeDtypeStruct((B, NUM_EXPERTS, C), jnp.float32),
        compiler_params=pltpu.CompilerParams(
            dimension_semantics=("parallel",)),
    )(f0, f0, f0, f0, f1, f1, f2, f3)

    feat = pooled.reshape(B, D)
    out, gw = pl.pallas_call(
        _head_body,
        in_specs=[
            pl.BlockSpec((B, D), lambda: (0, 0)),
            pl.BlockSpec(Wg.shape, lambda: (0, 0)),
            pl.BlockSpec((1, NUM_EXPERTS), lambda: (0, 0)),
            pl.BlockSpec(We.shape, lambda: (0, 0, 0)),
            pl.BlockSpec(be.shape, lambda: (0, 0)),
        ],
        out_specs=[
            pl.BlockSpec((B, NUM_CLASSES), lambda: (0, 0)),
            pl.BlockSpec((B, NUM_EXPERTS), lambda: (0, 0)),
        ],
        out_shape=[
            jax.ShapeDtypeStruct((B, NUM_CLASSES), jnp.float32),
            jax.ShapeDtypeStruct((B, NUM_EXPERTS), jnp.float32),
        ],
    )(feat, Wg, bg.reshape(1, NUM_EXPERTS), We, be)
    return (out, gw)
